# baseline (speedup 1.0000x reference)
import functools

import jax
import jax.numpy as jnp
from jax import lax
from jax.experimental import pallas as pl
from jax.experimental.pallas import tpu as pltpu
from jax.experimental.pallas import tpu_sc as plsc

VOCAB = 100000
EMBED_DIM = 64
CONTEXT = 4
BATCH = 1024
HIDDEN = 128

VT = 2048
NT = (VOCAB + VT - 1) // VT

LOG2E = 1.4426950408889634
LN2 = 0.6931471805599453
NEG_BIG = -1e30


def _stats_body(embeds_ref, w1_ref, b1_ref, w2_ref, b2_ref,
                ht_out, mls_out, h2t_ref, m_ref, s_ref):
    t = pl.program_id(0)

    @pl.when(t == 0)
    def _init():
        acc = None
        for c in range(CONTEXT):
            ec = embeds_ref[c][:, :EMBED_DIM].astype(jnp.bfloat16)
            w1c = w1_ref[:, c * EMBED_DIM:(c + 1) * EMBED_DIM]
            part = lax.dot_general(ec, w1c.astype(jnp.bfloat16),
                                   (((1,), (1,)), ((), ())),
                                   preferred_element_type=jnp.float32)
            acc = part if acc is None else acc + part
        hf = jnp.maximum(acc + b1_ref[...], 0.0)
        hft = hf.T
        ht_out[...] = hft.astype(jnp.bfloat16)
        h2t_ref[...] = (hft * LOG2E).astype(jnp.bfloat16)
        m_ref[...] = jnp.full_like(m_ref, NEG_BIG)
        s_ref[...] = jnp.zeros_like(s_ref)

    row = t * VT + lax.broadcasted_iota(jnp.int32, (VT, 1), 0)
    w2bf = jnp.where(row < VOCAB, w2_ref[...], 0.0).astype(jnp.bfloat16)
    b22 = jnp.where(row < VOCAB, b2_ref[...].T * LOG2E, NEG_BIG)
    lg2 = lax.dot_general(w2bf, h2t_ref[...], (((1,), (0,)), ((), ())),
                          preferred_element_type=jnp.float32) + b22
    tile_max = jnp.max(lg2, axis=0, keepdims=True)
    m_old = m_ref[...]
    m_new = jnp.maximum(m_old, tile_max)
    s_ref[...] = (s_ref[...] * jnp.exp2(m_old - m_new)
                  + jnp.sum(jnp.exp2(lg2 - m_new), axis=0, keepdims=True))
    m_ref[...] = m_new

    @pl.when(t == NT - 1)
    def _finalize():
        mls_out[...] = m_ref[...] * LN2 + jnp.log(s_ref[...])


def _write_body(ht_ref, w2_ref, b2_ref, mls_ref, out_ref):
    logits = lax.dot_general(w2_ref[...].astype(jnp.bfloat16), ht_ref[...],
                             (((1,), (0,)), ((), ())),
                             preferred_element_type=jnp.float32)
    out_ref[...] = (logits + b2_ref[...].T) - mls_ref[...]


def _fused_logsoftmax(e4, W1, b1, b2_row, W2, *, interpret=False):
    ht, mls = pl.pallas_call(
        _stats_body,
        grid=(NT,),
        in_specs=[
            pl.BlockSpec((CONTEXT, BATCH, 2 * EMBED_DIM), lambda t: (0, 0, 0)),
            pl.BlockSpec((HIDDEN, EMBED_DIM * CONTEXT), lambda t: (0, 0)),
            pl.BlockSpec((1, HIDDEN), lambda t: (0, 0)),
            pl.BlockSpec((VT, HIDDEN), lambda t: (t, 0)),
            pl.BlockSpec((1, VT), lambda t: (0, t)),
        ],
        out_specs=[
            pl.BlockSpec((HIDDEN, BATCH), lambda t: (0, 0)),
            pl.BlockSpec((1, BATCH), lambda t: (0, 0)),
        ],
        out_shape=[
            jax.ShapeDtypeStruct((HIDDEN, BATCH), jnp.bfloat16),
            jax.ShapeDtypeStruct((1, BATCH), jnp.float32),
        ],
        scratch_shapes=[
            pltpu.VMEM((HIDDEN, BATCH), jnp.bfloat16),
            pltpu.VMEM((1, BATCH), jnp.float32),
            pltpu.VMEM((1, BATCH), jnp.float32),
        ],
        compiler_params=pltpu.CompilerParams(
            dimension_semantics=("arbitrary",),
        ),
        interpret=interpret,
    )(e4, W1, b1, W2, b2_row)

    return pl.pallas_call(
        _write_body,
        grid=(NT,),
        in_specs=[
            pl.BlockSpec((HIDDEN, BATCH), lambda t: (0, 0)),
            pl.BlockSpec((VT, HIDDEN), lambda t: (t, 0)),
            pl.BlockSpec((1, VT), lambda t: (0, t)),
            pl.BlockSpec((1, BATCH), lambda t: (0, 0)),
        ],
        out_specs=pl.BlockSpec((VT, BATCH), lambda t: (t, 0)),
        out_shape=jax.ShapeDtypeStruct((VOCAB, BATCH), jnp.float32),
        compiler_params=pltpu.CompilerParams(
            dimension_semantics=("arbitrary",),
        ),
        interpret=interpret,
    )(ht, W2, b2_row, mls)


def _sc_gather(table, idx):
    n = idx.shape[0]
    d = table.shape[1]
    info = plsc.get_sparse_core_info()
    nw = info.num_cores * info.num_subcores
    b_per_w = n // nw
    mesh = plsc.VectorSubcoreMesh(core_axis_name="c", subcore_axis_name="s")

    @functools.partial(
        pl.kernel, mesh=mesh,
        out_type=jax.ShapeDtypeStruct((n, d), jnp.float32),
        scratch_types=[
            pltpu.VMEM((b_per_w,), jnp.int32),
            pltpu.VMEM((b_per_w, d), jnp.float32),
            pltpu.SemaphoreType.DMA,
        ],
    )
    def k(table_hbm, idx_hbm, out_hbm, idx_v, rows_v, sem):
        wid = lax.axis_index("s") * info.num_cores + lax.axis_index("c")
        base = wid * b_per_w
        pltpu.sync_copy(idx_hbm.at[pl.ds(base, b_per_w)], idx_v)
        pltpu.async_copy(table_hbm.at[idx_v], rows_v, sem).wait()
        pltpu.sync_copy(rows_v, out_hbm.at[pl.ds(base, b_per_w)])

    return k(table, idx)


def kernel(X, emb, W1, b1, W2, b2):
    idx = X.T.reshape(-1).astype(jnp.int32)
    embp = jnp.pad(emb, ((0, 0), (0, 2 * EMBED_DIM - emb.shape[1])))
    rows = _sc_gather(embp, idx)
    e4 = rows.reshape(CONTEXT, BATCH, 2 * EMBED_DIM)
    out_t = _fused_logsoftmax(e4, W1, b1.reshape(1, HIDDEN),
                              b2.reshape(1, VOCAB), W2)
    return out_t.T


# speedup vs baseline: 1.0158x; 1.0158x over previous
import functools

import jax
import jax.numpy as jnp
from jax import lax
from jax.experimental import pallas as pl
from jax.experimental.pallas import tpu as pltpu
from jax.experimental.pallas import tpu_sc as plsc

VOCAB = 100000
EMBED_DIM = 64
CONTEXT = 4
BATCH = 1024
HIDDEN = 128

VT = 2048
NT = (VOCAB + VT - 1) // VT

LOG2E = 1.4426950408889634
LN2 = 0.6931471805599453
NEG_BIG = -1e30


def _stats_body(embeds_ref, w1_ref, b1_ref, w2_ref, b2_ref,
                ht_out, mls_out, h2t_ref, m_ref, s_ref):
    t = pl.program_id(0)

    @pl.when(t == 0)
    def _init():
        acc = None
        for c in range(CONTEXT):
            ec = embeds_ref[c][:, :EMBED_DIM].astype(jnp.bfloat16)
            w1c = w1_ref[:, c * EMBED_DIM:(c + 1) * EMBED_DIM]
            part = lax.dot_general(ec, w1c.astype(jnp.bfloat16),
                                   (((1,), (1,)), ((), ())),
                                   preferred_element_type=jnp.float32)
            acc = part if acc is None else acc + part
        hf = jnp.maximum(acc + b1_ref[...], 0.0)
        hft = hf.T
        ht_out[...] = hft.astype(jnp.bfloat16)
        h2t_ref[...] = (hft * LOG2E).astype(jnp.bfloat16)
        m_ref[...] = jnp.full_like(m_ref, NEG_BIG)
        s_ref[...] = jnp.zeros_like(s_ref)

    row = t * VT + lax.broadcasted_iota(jnp.int32, (VT, 1), 0)
    w2bf = jnp.where(row < VOCAB, w2_ref[...], 0.0).astype(jnp.bfloat16)
    b22 = jnp.where(row < VOCAB, b2_ref[...].T * LOG2E,
                    NEG_BIG).astype(jnp.bfloat16)
    lg2 = lax.dot_general(w2bf, h2t_ref[...], (((1,), (0,)), ((), ())),
                          preferred_element_type=jnp.float32)
    xbf = lg2.astype(jnp.bfloat16) + b22
    tile_max = jnp.max(xbf, axis=0, keepdims=True).astype(jnp.float32)
    m_old = m_ref[...]
    m_new = jnp.maximum(m_old, tile_max)
    q = jnp.exp2(xbf - m_new.astype(jnp.bfloat16))
    ones_row = jnp.ones((1, VT), jnp.bfloat16)
    s_tile = lax.dot_general(ones_row, q, (((1,), (0,)), ((), ())),
                             preferred_element_type=jnp.float32)
    s_ref[...] = s_ref[...] * jnp.exp2(m_old - m_new) + s_tile
    m_ref[...] = m_new

    @pl.when(t == NT - 1)
    def _finalize():
        mls_out[...] = m_ref[...] * LN2 + jnp.log(s_ref[...])


def _write_body(ht_ref, w2_ref, b2_ref, mls_ref, out_ref):
    logits = lax.dot_general(w2_ref[...].astype(jnp.bfloat16), ht_ref[...],
                             (((1,), (0,)), ((), ())),
                             preferred_element_type=jnp.float32)
    out_ref[...] = (logits + b2_ref[...].T) - mls_ref[...]


def _fused_logsoftmax(e4, W1, b1, b2_row, W2, *, interpret=False):
    ht, mls = pl.pallas_call(
        _stats_body,
        grid=(NT,),
        in_specs=[
            pl.BlockSpec((CONTEXT, BATCH, 2 * EMBED_DIM), lambda t: (0, 0, 0)),
            pl.BlockSpec((HIDDEN, EMBED_DIM * CONTEXT), lambda t: (0, 0)),
            pl.BlockSpec((1, HIDDEN), lambda t: (0, 0)),
            pl.BlockSpec((VT, HIDDEN), lambda t: (t, 0)),
            pl.BlockSpec((1, VT), lambda t: (0, t)),
        ],
        out_specs=[
            pl.BlockSpec((HIDDEN, BATCH), lambda t: (0, 0)),
            pl.BlockSpec((1, BATCH), lambda t: (0, 0)),
        ],
        out_shape=[
            jax.ShapeDtypeStruct((HIDDEN, BATCH), jnp.bfloat16),
            jax.ShapeDtypeStruct((1, BATCH), jnp.float32),
        ],
        scratch_shapes=[
            pltpu.VMEM((HIDDEN, BATCH), jnp.bfloat16),
            pltpu.VMEM((1, BATCH), jnp.float32),
            pltpu.VMEM((1, BATCH), jnp.float32),
        ],
        compiler_params=pltpu.CompilerParams(
            dimension_semantics=("arbitrary",),
        ),
        interpret=interpret,
    )(e4, W1, b1, W2, b2_row)

    return pl.pallas_call(
        _write_body,
        grid=(NT,),
        in_specs=[
            pl.BlockSpec((HIDDEN, BATCH), lambda t: (0, 0)),
            pl.BlockSpec((VT, HIDDEN), lambda t: (t, 0)),
            pl.BlockSpec((1, VT), lambda t: (0, t)),
            pl.BlockSpec((1, BATCH), lambda t: (0, 0)),
        ],
        out_specs=pl.BlockSpec((VT, BATCH), lambda t: (t, 0)),
        out_shape=jax.ShapeDtypeStruct((VOCAB, BATCH), jnp.float32),
        compiler_params=pltpu.CompilerParams(
            dimension_semantics=("arbitrary",),
        ),
        interpret=interpret,
    )(ht, W2, b2_row, mls)


def _sc_gather(table, idx):
    n = idx.shape[0]
    d = table.shape[1]
    info = plsc.get_sparse_core_info()
    nw = info.num_cores * info.num_subcores
    b_per_w = n // nw
    mesh = plsc.VectorSubcoreMesh(core_axis_name="c", subcore_axis_name="s")

    @functools.partial(
        pl.kernel, mesh=mesh,
        out_type=jax.ShapeDtypeStruct((n, d), jnp.float32),
        scratch_types=[
            pltpu.VMEM((b_per_w,), jnp.int32),
            pltpu.VMEM((b_per_w, d), jnp.float32),
            pltpu.SemaphoreType.DMA,
        ],
    )
    def k(table_hbm, idx_hbm, out_hbm, idx_v, rows_v, sem):
        wid = lax.axis_index("s") * info.num_cores + lax.axis_index("c")
        base = wid * b_per_w
        pltpu.sync_copy(idx_hbm.at[pl.ds(base, b_per_w)], idx_v)
        pltpu.async_copy(table_hbm.at[idx_v], rows_v, sem).wait()
        pltpu.sync_copy(rows_v, out_hbm.at[pl.ds(base, b_per_w)])

    return k(table, idx)


def kernel(X, emb, W1, b1, W2, b2):
    idx = X.T.reshape(-1).astype(jnp.int32)
    embp = jnp.pad(emb, ((0, 0), (0, 2 * EMBED_DIM - emb.shape[1])))
    rows = _sc_gather(embp, idx)
    e4 = rows.reshape(CONTEXT, BATCH, 2 * EMBED_DIM)
    out_t = _fused_logsoftmax(e4, W1, b1.reshape(1, HIDDEN),
                              b2.reshape(1, VOCAB), W2)
    return out_t.T


# speedup vs baseline: 1.0707x; 1.0540x over previous
import functools

import jax
import jax.numpy as jnp
from jax import lax
from jax.experimental import pallas as pl
from jax.experimental.pallas import tpu as pltpu
from jax.experimental.pallas import tpu_sc as plsc

VOCAB = 100000
EMBED_DIM = 64
CONTEXT = 4
BATCH = 1024
HIDDEN = 128

VT = 2048
NT = (VOCAB + VT - 1) // VT

LOG2E = 1.4426950408889634
LN2 = 0.6931471805599453
NEG_BIG = -1e30


def _fused_body(embeds_ref, w1_ref, b1_ref, w2_ref, b2_ref, out_ref,
                ht_ref, h2t_ref, w2s_ref, m_ref, s_ref, mls_ref):
    p = pl.program_id(0)
    t = pl.program_id(1)

    @pl.when((p == 0) & (t == 0))
    def _init():
        acc = None
        for c in range(CONTEXT):
            ec = embeds_ref[c][:, :EMBED_DIM].astype(jnp.bfloat16)
            w1c = w1_ref[:, c * EMBED_DIM:(c + 1) * EMBED_DIM]
            part = lax.dot_general(ec, w1c.astype(jnp.bfloat16),
                                   (((1,), (1,)), ((), ())),
                                   preferred_element_type=jnp.float32)
            acc = part if acc is None else acc + part
        hf = jnp.maximum(acc + b1_ref[...], 0.0)
        hft = hf.T
        ht_ref[...] = hft.astype(jnp.bfloat16)
        h2t_ref[...] = (hft * LOG2E).astype(jnp.bfloat16)
        m_ref[...] = jnp.full_like(m_ref, NEG_BIG)
        s_ref[...] = jnp.zeros_like(s_ref)

    @pl.when(p == 0)
    def _stats():
        row = t * VT + lax.broadcasted_iota(jnp.int32, (VT, 1), 0)
        w2bf = jnp.where(row < VOCAB, w2_ref[...], 0.0).astype(jnp.bfloat16)
        w2s_ref[pl.ds(t * VT, VT), :] = w2bf
        b22 = jnp.where(row < VOCAB, b2_ref[...].T * LOG2E,
                        NEG_BIG).astype(jnp.bfloat16)
        lg2 = lax.dot_general(w2bf, h2t_ref[...], (((1,), (0,)), ((), ())),
                              preferred_element_type=jnp.float32)
        xbf = lg2.astype(jnp.bfloat16) + b22
        tile_max = jnp.max(xbf, axis=0, keepdims=True).astype(jnp.float32)
        m_old = m_ref[...]
        m_new = jnp.maximum(m_old, tile_max)
        q = jnp.exp2(xbf - m_new.astype(jnp.bfloat16))
        ones_row = jnp.ones((1, VT), jnp.bfloat16)
        s_tile = lax.dot_general(ones_row, q, (((1,), (0,)), ((), ())),
                                 preferred_element_type=jnp.float32)
        s_ref[...] = s_ref[...] * jnp.exp2(m_old - m_new) + s_tile
        m_ref[...] = m_new

    @pl.when(p == 1)
    def _write():
        @pl.when(t == 0)
        def _finalize():
            mls_ref[...] = m_ref[...] * LN2 + jnp.log(s_ref[...])

        w2bf = w2s_ref[pl.ds(t * VT, VT), :]
        logits = lax.dot_general(w2bf, ht_ref[...], (((1,), (0,)), ((), ())),
                                 preferred_element_type=jnp.float32)
        out_ref[...] = (logits + b2_ref[...].T) - mls_ref[...]


def _fused_logsoftmax(e4, W1, b1, b2_row, W2, *, interpret=False):
    return pl.pallas_call(
        _fused_body,
        grid=(2, NT),
        in_specs=[
            pl.BlockSpec((CONTEXT, BATCH, 2 * EMBED_DIM),
                         lambda p, t: (0, 0, 0)),
            pl.BlockSpec((HIDDEN, EMBED_DIM * CONTEXT), lambda p, t: (0, 0)),
            pl.BlockSpec((1, HIDDEN), lambda p, t: (0, 0)),
            pl.BlockSpec((VT, HIDDEN), lambda p, t: (t * (1 - p), 0)),
            pl.BlockSpec((1, VT), lambda p, t: (0, t)),
        ],
        out_specs=pl.BlockSpec((VT, BATCH), lambda p, t: (p * t, 0)),
        out_shape=jax.ShapeDtypeStruct((VOCAB, BATCH), jnp.float32),
        scratch_shapes=[
            pltpu.VMEM((HIDDEN, BATCH), jnp.bfloat16),
            pltpu.VMEM((HIDDEN, BATCH), jnp.bfloat16),
            pltpu.VMEM((NT * VT, HIDDEN), jnp.bfloat16),
            pltpu.VMEM((1, BATCH), jnp.float32),
            pltpu.VMEM((1, BATCH), jnp.float32),
            pltpu.VMEM((1, BATCH), jnp.float32),
        ],
        compiler_params=pltpu.CompilerParams(
            dimension_semantics=("arbitrary", "arbitrary"),
        ),
        interpret=interpret,
    )(e4, W1, b1, W2, b2_row)


def _sc_gather(table, idx):
    n = idx.shape[0]
    d = table.shape[1]
    info = plsc.get_sparse_core_info()
    nw = info.num_cores * info.num_subcores
    b_per_w = n // nw
    mesh = plsc.VectorSubcoreMesh(core_axis_name="c", subcore_axis_name="s")

    @functools.partial(
        pl.kernel, mesh=mesh,
        out_type=jax.ShapeDtypeStruct((n, d), jnp.float32),
        scratch_types=[
            pltpu.VMEM((b_per_w,), jnp.int32),
            pltpu.VMEM((b_per_w, d), jnp.float32),
            pltpu.SemaphoreType.DMA,
        ],
    )
    def k(table_hbm, idx_hbm, out_hbm, idx_v, rows_v, sem):
        wid = lax.axis_index("s") * info.num_cores + lax.axis_index("c")
        base = wid * b_per_w
        pltpu.sync_copy(idx_hbm.at[pl.ds(base, b_per_w)], idx_v)
        pltpu.async_copy(table_hbm.at[idx_v], rows_v, sem).wait()
        pltpu.sync_copy(rows_v, out_hbm.at[pl.ds(base, b_per_w)])

    return k(table, idx)


def kernel(X, emb, W1, b1, W2, b2):
    idx = X.T.reshape(-1).astype(jnp.int32)
    embp = jnp.pad(emb, ((0, 0), (0, 2 * EMBED_DIM - emb.shape[1])))
    rows = _sc_gather(embp, idx)
    e4 = rows.reshape(CONTEXT, BATCH, 2 * EMBED_DIM)
    out_t = _fused_logsoftmax(e4, W1, b1.reshape(1, HIDDEN),
                              b2.reshape(1, VOCAB), W2)
    return out_t.T


# speedup vs baseline: 1.0944x; 1.0221x over previous
import functools

import jax
import jax.numpy as jnp
from jax import lax
from jax.experimental import pallas as pl
from jax.experimental.pallas import tpu as pltpu
from jax.experimental.pallas import tpu_sc as plsc

VOCAB = 100000
EMBED_DIM = 64
CONTEXT = 4
BATCH = 1024
HIDDEN = 128

VT = 2048
NT = (VOCAB + VT - 1) // VT

LOG2E = 1.4426950408889634
LN2 = 0.6931471805599453
NEG_BIG = -1e30


def _fused_body(embeds_ref, w1_ref, b1_ref, w2_ref, b2_ref, out_ref,
                ht_ref, h2t_ref, w2s_ref, m_ref, s_ref, mls_ref):
    p = pl.program_id(0)
    t = pl.program_id(1)

    @pl.when((p == 0) & (t == 0))
    def _init():
        acc = None
        for c in range(CONTEXT):
            ec = embeds_ref[c][:, :EMBED_DIM].astype(jnp.bfloat16)
            w1c = w1_ref[:, c * EMBED_DIM:(c + 1) * EMBED_DIM]
            part = lax.dot_general(ec, w1c.astype(jnp.bfloat16),
                                   (((1,), (1,)), ((), ())),
                                   preferred_element_type=jnp.float32)
            acc = part if acc is None else acc + part
        hf = jnp.maximum(acc + b1_ref[...], 0.0)
        hft = hf.T
        ht_ref[...] = hft.astype(jnp.bfloat16)
        h2t_ref[...] = (hft * LOG2E).astype(jnp.bfloat16)
        m_ref[...] = jnp.full_like(m_ref, NEG_BIG)
        s_ref[...] = jnp.zeros_like(s_ref)

    @pl.when(p == 0)
    def _stats():
        row = t * VT + lax.broadcasted_iota(jnp.int32, (VT, 1), 0)
        w2bf = jnp.where(row < VOCAB, w2_ref[...], 0.0).astype(jnp.bfloat16)
        w2s_ref[pl.ds(t * VT, VT), :] = w2bf
        b22 = jnp.where(row < VOCAB, b2_ref[...].T * LOG2E,
                        NEG_BIG).astype(jnp.bfloat16)
        lg2 = lax.dot_general(w2bf, h2t_ref[...], (((1,), (0,)), ((), ())),
                              preferred_element_type=jnp.float32)
        xbf = lg2.astype(jnp.bfloat16) + b22
        tile_max = jnp.max(xbf, axis=0, keepdims=True).astype(jnp.float32)
        m_old = m_ref[...]
        m_new = jnp.maximum(m_old, tile_max)
        q = jnp.exp2(xbf - m_new.astype(jnp.bfloat16))
        ones_row = jnp.ones((1, VT), jnp.bfloat16)
        s_tile = lax.dot_general(ones_row, q, (((1,), (0,)), ((), ())),
                                 preferred_element_type=jnp.float32)
        s_ref[...] = s_ref[...] * jnp.exp2(m_old - m_new) + s_tile
        m_ref[...] = m_new

    @pl.when(p == 1)
    def _write():
        @pl.when(t == 0)
        def _finalize():
            mls_ref[...] = m_ref[...] * LN2 + jnp.log(s_ref[...])

        w2bf = w2s_ref[pl.ds(t * VT, VT), :]
        logits = lax.dot_general(w2bf, ht_ref[...], (((1,), (0,)), ((), ())),
                                 preferred_element_type=jnp.float32)
        out_ref[...] = (logits + b2_ref[...].T) - mls_ref[...]


def _fused_logsoftmax(e4, W1, b1, b2_row, W2, *, interpret=False):
    return pl.pallas_call(
        _fused_body,
        grid=(2, NT),
        in_specs=[
            pl.BlockSpec((CONTEXT, BATCH, 2 * EMBED_DIM),
                         lambda p, t: (0, 0, 0)),
            pl.BlockSpec((HIDDEN, EMBED_DIM * CONTEXT), lambda p, t: (0, 0)),
            pl.BlockSpec((1, HIDDEN), lambda p, t: (0, 0)),
            pl.BlockSpec((VT, HIDDEN), lambda p, t: (t * (1 - p), 0)),
            pl.BlockSpec((1, VT), lambda p, t: (0, t)),
        ],
        out_specs=pl.BlockSpec((VT, BATCH), lambda p, t: (p * t, 0)),
        out_shape=jax.ShapeDtypeStruct((VOCAB, BATCH), jnp.float32),
        scratch_shapes=[
            pltpu.VMEM((HIDDEN, BATCH), jnp.bfloat16),
            pltpu.VMEM((HIDDEN, BATCH), jnp.bfloat16),
            pltpu.VMEM((NT * VT, HIDDEN), jnp.bfloat16),
            pltpu.VMEM((1, BATCH), jnp.float32),
            pltpu.VMEM((1, BATCH), jnp.float32),
            pltpu.VMEM((1, BATCH), jnp.float32),
        ],
        compiler_params=pltpu.CompilerParams(
            dimension_semantics=("arbitrary", "arbitrary"),
        ),
        interpret=interpret,
    )(e4, W1, b1, W2, b2_row)


TPT = 2048


def _tp_body(embt_ref, out_ref):
    xt = embt_ref[...].T
    out_ref[...] = jnp.concatenate(
        [xt, jnp.zeros((TPT, 2 * EMBED_DIM - EMBED_DIM), jnp.float32)],
        axis=1)


def _transpose_pad(embt):
    nt = (VOCAB + TPT - 1) // TPT
    return pl.pallas_call(
        _tp_body,
        grid=(nt,),
        in_specs=[pl.BlockSpec((EMBED_DIM, TPT), lambda t: (0, t))],
        out_specs=pl.BlockSpec((TPT, 2 * EMBED_DIM), lambda t: (t, 0)),
        out_shape=jax.ShapeDtypeStruct((nt * TPT, 2 * EMBED_DIM),
                                       jnp.float32),
        compiler_params=pltpu.CompilerParams(
            dimension_semantics=("arbitrary",),
        ),
    )(embt)


def _sc_gather(table, idx):
    n = idx.shape[0]
    d = table.shape[1]
    info = plsc.get_sparse_core_info()
    nw = info.num_cores * info.num_subcores
    b_per_w = n // nw
    mesh = plsc.VectorSubcoreMesh(core_axis_name="c", subcore_axis_name="s")

    @functools.partial(
        pl.kernel, mesh=mesh,
        out_type=jax.ShapeDtypeStruct((n, d), jnp.float32),
        scratch_types=[
            pltpu.VMEM((b_per_w,), jnp.int32),
            pltpu.VMEM((b_per_w, d), jnp.float32),
            pltpu.SemaphoreType.DMA,
        ],
    )
    def k(table_hbm, idx_hbm, out_hbm, idx_v, rows_v, sem):
        wid = lax.axis_index("s") * info.num_cores + lax.axis_index("c")
        base = wid * b_per_w
        pltpu.sync_copy(idx_hbm.at[pl.ds(base, b_per_w)], idx_v)
        pltpu.async_copy(table_hbm.at[idx_v], rows_v, sem).wait()
        pltpu.sync_copy(rows_v, out_hbm.at[pl.ds(base, b_per_w)])

    return k(table, idx)


def kernel(X, emb, W1, b1, W2, b2):
    idx = X.T.reshape(-1).astype(jnp.int32)
    embp = _transpose_pad(emb.T)
    rows = _sc_gather(embp, idx)
    e4 = rows.reshape(CONTEXT, BATCH, 2 * EMBED_DIM)
    out_t = _fused_logsoftmax(e4, W1, b1.reshape(1, HIDDEN),
                              b2.reshape(1, VOCAB), W2)
    return out_t.T
